# Initial kernel scaffold; baseline (speedup 1.0000x reference)
import jax
import jax.numpy as jnp
from jax.experimental import pallas as pl


def kernel(x, wpe):
    raise NotImplementedError("write your pallas kernel here")


# speedup vs baseline: 1.6726x; 1.6726x over previous
import jax
import jax.numpy as jnp
from jax.experimental import pallas as pl

BT = 512


def _body(x_ref, wpe_ref, o_ref):
    o_ref[0] = x_ref[0] + wpe_ref[...]


def kernel(x, wpe):
    b, t, d = x.shape
    grid = (t // BT, b)
    return pl.pallas_call(
        _body,
        grid=grid,
        in_specs=[
            pl.BlockSpec((1, BT, d), lambda i, j: (j, i, 0)),
            pl.BlockSpec((BT, d), lambda i, j: (i, 0)),
        ],
        out_specs=pl.BlockSpec((1, BT, d), lambda i, j: (j, i, 0)),
        out_shape=jax.ShapeDtypeStruct((b, t, d), x.dtype),
    )(x, wpe)
